# baseline (speedup 1.0000x reference)
import functools

import jax
import jax.numpy as jnp
from jax import lax
from jax.experimental import pallas as pl
from jax.experimental.pallas import tpu as pltpu
from jax.experimental.pallas import tpu_sc as plsc


def _tc_scores_body(xq_ref, ek_ref, idx_out):
    xq = xq_ref[...]
    ek = ek_ref[...]
    nk = ek / jnp.maximum(
        jnp.sqrt(jnp.sum(ek * ek, axis=1, keepdims=True)), 1e-12)
    nq = xq / jnp.maximum(
        jnp.sqrt(jnp.sum(xq * xq, axis=1, keepdims=True)), 1e-12)
    scores = jax.lax.dot_general(nq, nk, (((1,), (1,)), ((), ())))
    idx_out[...] = jnp.argmax(scores, axis=1).astype(jnp.int32)[:, None]


def _tc_scores(x_querry, e_k, blk=512):
    b, key_d = x_querry.shape
    pool, _ = e_k.shape
    grid = (b // blk,)
    return pl.pallas_call(
        _tc_scores_body,
        grid=grid,
        in_specs=[
            pl.BlockSpec((blk, key_d), lambda i: (i, 0)),
            pl.BlockSpec((pool, key_d), lambda i: (0, 0)),
        ],
        out_specs=[
            pl.BlockSpec((blk, 1), lambda i: (i, 0)),
        ],
        out_shape=[
            jax.ShapeDtypeStruct((b, 1), jnp.int32),
        ],
    )(x_querry, e_k)


def _sc_gather(e_p, idx, nc, ns, ch=8):
    b = idx.shape[0]
    pool, p_len, emb_d = e_p.shape
    half = p_len // 2
    nw = nc * ns
    bw = b // nw
    nch = bw // ch
    mesh = plsc.VectorSubcoreMesh(core_axis_name="c", subcore_axis_name="s")

    @functools.partial(
        pl.kernel,
        out_type=[
            jax.ShapeDtypeStruct((b, half, emb_d), jnp.float32),
            jax.ShapeDtypeStruct((b, half, emb_d), jnp.float32),
        ],
        mesh=mesh,
        scratch_types=[
            pltpu.VMEM((bw,), jnp.int32),
            pltpu.VMEM((2, ch, p_len, emb_d), jnp.float32),
            pltpu.SemaphoreType.DMA,
            pltpu.SemaphoreType.DMA,
            pltpu.SemaphoreType.DMA,
            pltpu.SemaphoreType.DMA,
            pltpu.SemaphoreType.DMA,
            pltpu.SemaphoreType.DMA,
        ],
    )
    def k(ep_hbm, idx_hbm, ek_out, ev_out,
          idx_v, buf, g0, g1, ok0, ok1, ov0, ov1):
        wid = lax.axis_index("s") * nc + lax.axis_index("c")
        base = wid * bw
        pltpu.sync_copy(idx_hbm.at[pl.ds(base, bw)], idx_v)
        gsem = [g0, g1]
        osem = [(ok0, ov0), (ok1, ov1)]

        def fire(c):
            s = c % 2
            iref = idx_v.at[pl.ds(c * ch, ch)]
            return pltpu.async_copy(ep_hbm.at[iref], buf.at[s], gsem[s])

        gh = [None] * nch
        oh = [None] * nch
        gh[0] = fire(0)
        for c in range(nch):
            s = c % 2
            if c + 1 < nch:
                if c >= 1:
                    oh[c - 1][0].wait()
                    oh[c - 1][1].wait()
                gh[c + 1] = fire(c + 1)
            gh[c].wait()
            dst = pl.ds(base + c * ch, ch)
            oh[c] = (
                pltpu.async_copy(buf.at[s, :, pl.ds(0, half)],
                                 ek_out.at[dst], osem[s][0]),
                pltpu.async_copy(buf.at[s, :, pl.ds(half, half)],
                                 ev_out.at[dst], osem[s][1]),
            )
        for c in (nch - 2, nch - 1):
            oh[c][0].wait()
            oh[c][1].wait()

    return k(e_p, idx)


def kernel(x_querry, l, x_block, e_p, e_k):
    b = x_querry.shape[0]

    (idx2d,) = _tc_scores(x_querry, e_k)
    idx = idx2d.reshape(b)

    info = plsc.get_sparse_core_info()
    ekf, evf = _sc_gather(e_p, idx, info.num_cores, info.num_subcores)
    return (ekf, evf, x_block)


# speedup vs baseline: 1.0308x; 1.0308x over previous
import functools

import jax
import jax.numpy as jnp
from jax import lax
from jax.experimental import pallas as pl
from jax.experimental.pallas import tpu as pltpu
from jax.experimental.pallas import tpu_sc as plsc


def _tc_scores_body(xq_ref, ek_ref, xb_ref, idx_out, xb_out):
    xq = xq_ref[...]
    ek = ek_ref[...]
    nk = ek / jnp.maximum(
        jnp.sqrt(jnp.sum(ek * ek, axis=1, keepdims=True)), 1e-12)
    nq = xq / jnp.maximum(
        jnp.sqrt(jnp.sum(xq * xq, axis=1, keepdims=True)), 1e-12)
    scores = jax.lax.dot_general(nq, nk, (((1,), (1,)), ((), ())))
    idx_out[...] = jnp.argmax(scores, axis=1).astype(jnp.int32)[:, None]
    xb_out[...] = xb_ref[...]


def _tc_scores(x_querry, x_block, e_k, blk=512):
    b, key_d = x_querry.shape
    pool, _ = e_k.shape
    emb_d = x_block.shape[1]
    grid = (b // blk,)
    return pl.pallas_call(
        _tc_scores_body,
        grid=grid,
        in_specs=[
            pl.BlockSpec((blk, key_d), lambda i: (i, 0)),
            pl.BlockSpec((pool, key_d), lambda i: (0, 0)),
            pl.BlockSpec((blk, emb_d), lambda i: (i, 0)),
        ],
        out_specs=[
            pl.BlockSpec((blk, 1), lambda i: (i, 0)),
            pl.BlockSpec((blk, emb_d), lambda i: (i, 0)),
        ],
        out_shape=[
            jax.ShapeDtypeStruct((b, 1), jnp.int32),
            jax.ShapeDtypeStruct((b, emb_d), jnp.float32),
        ],
    )(x_querry, e_k, x_block)


def _sc_gather(e_p, idx, nc, ns, ch=8):
    b = idx.shape[0]
    pool, p_len, emb_d = e_p.shape
    half = p_len // 2
    nw = nc * ns
    bw = b // nw
    nch = bw // ch
    mesh = plsc.VectorSubcoreMesh(core_axis_name="c", subcore_axis_name="s")

    nb = 5

    @functools.partial(
        pl.kernel,
        out_type=[
            jax.ShapeDtypeStruct((b, half, emb_d), jnp.float32),
            jax.ShapeDtypeStruct((b, half, emb_d), jnp.float32),
        ],
        mesh=mesh,
        scratch_types=[
            pltpu.VMEM((bw,), jnp.int32),
            pltpu.VMEM((nb, ch, half, emb_d), jnp.float32),
            pltpu.SemaphoreType.DMA,
            pltpu.SemaphoreType.DMA,
            pltpu.SemaphoreType.DMA,
            pltpu.SemaphoreType.DMA,
            pltpu.SemaphoreType.DMA,
            pltpu.SemaphoreType.DMA,
            pltpu.SemaphoreType.DMA,
            pltpu.SemaphoreType.DMA,
            pltpu.SemaphoreType.DMA,
            pltpu.SemaphoreType.DMA,
        ],
    )
    def k(ep_hbm, idx_hbm, ek_out, ev_out,
          idx_v, bufs, g0, g1, g2, g3, g4, o0, o1, o2, o3, o4):
        wid = lax.axis_index("s") * nc + lax.axis_index("c")
        base = wid * bw

        pltpu.sync_copy(idx_hbm.at[pl.ds(base, bw)], idx_v)

        gsem = [g0, g1, g2, g3, g4]
        osem = [o0, o1, o2, o3, o4]
        jobs = 2 * nch

        def fire(j):
            c, h = divmod(j, 2)
            iref = idx_v.at[pl.ds(c * ch, ch)]
            src = ep_hbm.at[iref, pl.ds(h * half, half)]
            return pltpu.async_copy(src, bufs.at[j % nb], gsem[j % nb])

        gh = [None] * jobs
        oh = [None] * jobs
        for j in range(min(nb, jobs)):
            gh[j] = fire(j)
        for j in range(jobs):
            r = j % nb
            gh[j].wait()
            c, h = divmod(j, 2)
            dst = (ek_out if h == 0 else ev_out).at[pl.ds(base + c * ch, ch)]
            oh[j] = pltpu.async_copy(bufs.at[r], dst, osem[r])
            if j + nb < jobs:
                oh[j].wait()
                gh[j + nb] = fire(j + nb)
        for j in range(max(0, jobs - nb), jobs):
            oh[j].wait()

    return k(e_p, idx)


def kernel(x_querry, l, x_block, e_p, e_k):
    b = x_querry.shape[0]

    idx2d, xb_out = _tc_scores(x_querry, x_block, e_k)
    idx = idx2d.reshape(b)

    info = plsc.get_sparse_core_info()
    ekf, evf = _sc_gather(e_p, idx, info.num_cores, info.num_subcores)
    return (ekf, evf, xb_out)


# speedup vs baseline: 1.0547x; 1.0232x over previous
import functools

import jax
import jax.numpy as jnp
from jax import lax
from jax.experimental import pallas as pl
from jax.experimental.pallas import tpu as pltpu
from jax.experimental.pallas import tpu_sc as plsc


def _tc_scores_body(xq_ref, ek_ref, xb_ref, idx_out, xb_out):
    xq = xq_ref[...]
    ek = ek_ref[...]
    nk = ek / jnp.maximum(
        jnp.sqrt(jnp.sum(ek * ek, axis=1, keepdims=True)), 1e-12)
    nq = xq / jnp.maximum(
        jnp.sqrt(jnp.sum(xq * xq, axis=1, keepdims=True)), 1e-12)
    scores = jax.lax.dot_general(nq, nk, (((1,), (1,)), ((), ())))
    idx_out[...] = jnp.argmax(scores, axis=1).astype(jnp.int32)
    xb_out[...] = xb_ref[...]


def _tc_scores(x_querry, x_block, e_k, blk=512):
    b, key_d = x_querry.shape
    pool, _ = e_k.shape
    emb_d = x_block.shape[1]
    grid = (b // blk,)
    return pl.pallas_call(
        _tc_scores_body,
        grid=grid,
        in_specs=[
            pl.BlockSpec((blk, key_d), lambda i: (i, 0)),
            pl.BlockSpec((pool, key_d), lambda i: (0, 0)),
            pl.BlockSpec((blk, emb_d), lambda i: (i, 0)),
        ],
        out_specs=[
            pl.BlockSpec((blk,), lambda i: (i,)),
            pl.BlockSpec((blk, emb_d), lambda i: (i, 0)),
        ],
        out_shape=[
            jax.ShapeDtypeStruct((b,), jnp.int32),
            jax.ShapeDtypeStruct((b, emb_d), jnp.float32),
        ],
    )(x_querry, e_k, x_block)


def _sc_gather(e_p, idx, nc, ns, ch=8):
    b = idx.shape[0]
    pool, p_len, emb_d = e_p.shape
    half = p_len // 2
    nw = nc * ns
    bw = b // nw
    nch = bw // ch
    mesh = plsc.VectorSubcoreMesh(core_axis_name="c", subcore_axis_name="s")

    nb = 5

    @functools.partial(
        pl.kernel,
        out_type=[
            jax.ShapeDtypeStruct((b, half, emb_d), jnp.float32),
            jax.ShapeDtypeStruct((b, half, emb_d), jnp.float32),
        ],
        mesh=mesh,
        scratch_types=[
            pltpu.VMEM((bw,), jnp.int32),
            pltpu.VMEM((nb, ch, half, emb_d), jnp.float32),
            pltpu.SemaphoreType.DMA,
            pltpu.SemaphoreType.DMA,
            pltpu.SemaphoreType.DMA,
            pltpu.SemaphoreType.DMA,
            pltpu.SemaphoreType.DMA,
            pltpu.SemaphoreType.DMA,
            pltpu.SemaphoreType.DMA,
            pltpu.SemaphoreType.DMA,
            pltpu.SemaphoreType.DMA,
            pltpu.SemaphoreType.DMA,
        ],
    )
    def k(ep_hbm, idx_hbm, ek_out, ev_out,
          idx_v, bufs, g0, g1, g2, g3, g4, o0, o1, o2, o3, o4):
        wid = lax.axis_index("s") * nc + lax.axis_index("c")
        base = wid * bw

        pltpu.sync_copy(idx_hbm.at[pl.ds(base, bw)], idx_v)

        gsem = [g0, g1, g2, g3, g4]
        osem = [o0, o1, o2, o3, o4]
        jobs = 2 * nch

        def fire(j):
            c, h = divmod(j, 2)
            iref = idx_v.at[pl.ds(c * ch, ch)]
            src = ep_hbm.at[iref, pl.ds(h * half, half)]
            return pltpu.async_copy(src, bufs.at[j % nb], gsem[j % nb])

        gh = [None] * jobs
        oh = [None] * jobs
        for j in range(min(nb, jobs)):
            gh[j] = fire(j)
        for j in range(jobs):
            r = j % nb
            gh[j].wait()
            c, h = divmod(j, 2)
            dst = (ek_out if h == 0 else ev_out).at[pl.ds(base + c * ch, ch)]
            oh[j] = pltpu.async_copy(bufs.at[r], dst, osem[r])
            if j + nb < jobs:
                oh[j].wait()
                gh[j + nb] = fire(j + nb)
        for j in range(max(0, jobs - nb), jobs):
            oh[j].wait()

    return k(e_p, idx)


def kernel(x_querry, l, x_block, e_p, e_k):
    b = x_querry.shape[0]

    idx, xb_out = _tc_scores(x_querry, x_block, e_k)

    info = plsc.get_sparse_core_info()
    ekf, evf = _sc_gather(e_p, idx, info.num_cores, info.num_subcores)
    return (ekf, evf, xb_out)
